# initial kernel scaffold (unmeasured)
import jax
import jax.numpy as jnp
from jax import lax
from jax.experimental import pallas as pl
from jax.experimental.pallas import tpu as pltpu

N_DEV = 8


def kernel(x, w_mat):
    k_full, k_shard = x.shape
    _, n = w_mat.shape
    m_per = k_full // N_DEV

    def body(x_ref, w_ref, out_ref, xg_ref, amax_ref,
             send_sems, recv_sems, a_send_sems, a_recv_sems):
        my = lax.axis_index("i")

        sends = []
        for d in range(1, N_DEV):
            dst = lax.rem(my + d, N_DEV)
            rdma = pltpu.make_async_remote_copy(
                src_ref=x_ref.at[pl.ds(dst * m_per, m_per), :],
                dst_ref=xg_ref.at[:, pl.ds(my * k_shard, k_shard)],
                send_sem=send_sems.at[d],
                recv_sem=recv_sems.at[my],
                device_id=(dst,),
                device_id_type=pl.DeviceIdType.MESH,
            )
            rdma.start()
            sends.append(rdma)

        xg_ref[:, pl.ds(my * k_shard, k_shard)] = x_ref[pl.ds(my * m_per, m_per), :]

        for d in range(1, N_DEV):
            src = lax.rem(my + d, N_DEV)
            recv = pltpu.make_async_remote_copy(
                src_ref=x_ref.at[pl.ds(0, m_per), :],
                dst_ref=xg_ref.at[:, pl.ds(src * k_shard, k_shard)],
                send_sem=send_sems.at[d],
                recv_sem=recv_sems.at[src],
                device_id=(my,),
                device_id_type=pl.DeviceIdType.MESH,
            )
            recv.wait_recv()

        y = jnp.dot(xg_ref[...], w_ref[...], preferred_element_type=jnp.float32)
        y = jnp.maximum(y, 0.0)
        out_ref[...] = y
        amax = jnp.max(y)

        amax_ref[pl.ds(my, 1), :] = jnp.full((1, 128), amax, jnp.float32)
        a_sends = []
        for d in range(1, N_DEV):
            dst = lax.rem(my + d, N_DEV)
            rdma = pltpu.make_async_remote_copy(
                src_ref=amax_ref.at[pl.ds(my, 1), :],
                dst_ref=amax_ref.at[pl.ds(my, 1), :],
                send_sem=a_send_sems.at[d],
                recv_sem=a_recv_sems.at[my],
                device_id=(dst,),
                device_id_type=pl.DeviceIdType.MESH,
            )
            rdma.start()
            a_sends.append(rdma)
        for d in range(1, N_DEV):
            src = lax.rem(my + d, N_DEV)
            recv = pltpu.make_async_remote_copy(
                src_ref=amax_ref.at[pl.ds(src, 1), :],
                dst_ref=amax_ref.at[pl.ds(src, 1), :],
                send_sem=a_send_sems.at[d],
                recv_sem=a_recv_sems.at[src],
                device_id=(my,),
                device_id_type=pl.DeviceIdType.MESH,
            )
            recv.wait_recv()

        gmax = jnp.max(amax_ref[:, 0])
        scale = gmax / 127.0
        q = jnp.clip(jnp.round(out_ref[...] / scale), -127.0, 127.0)
        out_ref[...] = q * scale

        for r in sends:
            r.wait_send()
        for r in a_sends:
            r.wait_send()

    return pl.pallas_call(
        body,
        out_shape=jax.ShapeDtypeStruct((m_per, n), jnp.float32),
        in_specs=[
            pl.BlockSpec(memory_space=pltpu.VMEM),
            pl.BlockSpec(memory_space=pltpu.VMEM),
        ],
        out_specs=pl.BlockSpec(memory_space=pltpu.VMEM),
        scratch_shapes=[
            pltpu.VMEM((m_per, k_full), jnp.bfloat16),
            pltpu.VMEM((N_DEV, 128), jnp.float32),
            pltpu.SemaphoreType.DMA((N_DEV,)),
            pltpu.SemaphoreType.DMA((N_DEV,)),
            pltpu.SemaphoreType.DMA((N_DEV,)),
            pltpu.SemaphoreType.DMA((N_DEV,)),
        ],
        compiler_params=pltpu.CompilerParams(collective_id=0),
    )(x, w_mat)


# baseline (device time: 153527 ns/iter reference)
import jax
import jax.numpy as jnp
from jax import lax
from jax.experimental import pallas as pl
from jax.experimental.pallas import tpu as pltpu

N_DEV = 8
NB = 2048


def kernel(x, w_mat):
    k_full, k_shard = x.shape
    _, n = w_mat.shape
    m_per = k_full // N_DEV

    def body(x_ref, w_ref, out_ref, xg_ref, wbuf_ref, amax_ref,
             wdma_sems, send_sems, recv_sems, a_send_sems, a_recv_sems):
        my = lax.axis_index("i")

        sends = []
        for d in range(1, N_DEV):
            dst = lax.rem(my + d, N_DEV)
            rdma = pltpu.make_async_remote_copy(
                src_ref=x_ref.at[pl.ds(dst * m_per, m_per), :],
                dst_ref=xg_ref.at[:, pl.ds(my * k_shard, k_shard)],
                send_sem=send_sems.at[d],
                recv_sem=recv_sems.at[my],
                device_id=(dst,),
                device_id_type=pl.DeviceIdType.MESH,
            )
            rdma.start()
            sends.append(rdma)

        xg_ref[:, pl.ds(my * k_shard, k_shard)] = x_ref[pl.ds(my * m_per, m_per), :]

        def w_dma(d_next, slot):
            src = lax.rem(my + d_next, N_DEV)
            return pltpu.make_async_copy(
                w_ref.at[pl.ds(src * k_shard, k_shard), :],
                wbuf_ref.at[slot],
                wdma_sems.at[slot],
            )

        w_dma(0, 0).start()
        for d in range(N_DEV):
            if d + 1 < N_DEV:
                w_dma(d + 1, (d + 1) % 2).start()
            src = lax.rem(my + d, N_DEV)
            if d > 0:
                recv = pltpu.make_async_remote_copy(
                    src_ref=x_ref.at[pl.ds(0, m_per), :],
                    dst_ref=xg_ref.at[:, pl.ds(src * k_shard, k_shard)],
                    send_sem=send_sems.at[d],
                    recv_sem=recv_sems.at[src],
                    device_id=(my,),
                    device_id_type=pl.DeviceIdType.MESH,
                )
                recv.wait_recv()
            w_dma(d, d % 2).wait()
            xb = xg_ref[:, pl.ds(src * k_shard, k_shard)]
            for nb in range(0, n, NB):
                partial = jnp.dot(
                    xb,
                    wbuf_ref[d % 2, :, nb:nb + NB],
                    preferred_element_type=jnp.float32,
                )
                if d == 0:
                    out_ref[:, nb:nb + NB] = partial
                else:
                    out_ref[:, nb:nb + NB] += partial

        amax = jnp.float32(0.0)
        for nb in range(0, n, NB):
            y = jnp.maximum(out_ref[:, nb:nb + NB], 0.0)
            out_ref[:, nb:nb + NB] = y
            amax = jnp.maximum(amax, jnp.max(y))

        amax_ref[pl.ds(my, 1), :] = jnp.full((1, 128), amax, jnp.float32)
        a_sends = []
        for d in range(1, N_DEV):
            dst = lax.rem(my + d, N_DEV)
            rdma = pltpu.make_async_remote_copy(
                src_ref=amax_ref.at[pl.ds(my, 1), :],
                dst_ref=amax_ref.at[pl.ds(my, 1), :],
                send_sem=a_send_sems.at[d],
                recv_sem=a_recv_sems.at[my],
                device_id=(dst,),
                device_id_type=pl.DeviceIdType.MESH,
            )
            rdma.start()
            a_sends.append(rdma)
        for d in range(1, N_DEV):
            src = lax.rem(my + d, N_DEV)
            recv = pltpu.make_async_remote_copy(
                src_ref=amax_ref.at[pl.ds(src, 1), :],
                dst_ref=amax_ref.at[pl.ds(src, 1), :],
                send_sem=a_send_sems.at[d],
                recv_sem=a_recv_sems.at[src],
                device_id=(my,),
                device_id_type=pl.DeviceIdType.MESH,
            )
            recv.wait_recv()

        gmax = jnp.max(amax_ref[:, 0])
        scale = gmax / 127.0
        for nb in range(0, n, NB):
            q = jnp.clip(jnp.round(out_ref[:, nb:nb + NB] / scale), -127.0, 127.0)
            out_ref[:, nb:nb + NB] = q * scale

        for r in sends:
            r.wait_send()
        for r in a_sends:
            r.wait_send()

    return pl.pallas_call(
        body,
        out_shape=jax.ShapeDtypeStruct((m_per, n), jnp.float32),
        in_specs=[
            pl.BlockSpec(memory_space=pltpu.VMEM),
            pl.BlockSpec(memory_space=pl.ANY),
        ],
        out_specs=pl.BlockSpec(memory_space=pltpu.VMEM),
        scratch_shapes=[
            pltpu.VMEM((m_per, k_full), jnp.bfloat16),
            pltpu.VMEM((2, k_shard, n), jnp.bfloat16),
            pltpu.VMEM((N_DEV, 128), jnp.float32),
            pltpu.SemaphoreType.DMA((2,)),
            pltpu.SemaphoreType.DMA((N_DEV,)),
            pltpu.SemaphoreType.DMA((N_DEV,)),
            pltpu.SemaphoreType.DMA((N_DEV,)),
            pltpu.SemaphoreType.DMA((N_DEV,)),
        ],
    )(x.astype(jnp.bfloat16), w_mat.astype(jnp.bfloat16))


# device time: 115221 ns/iter; 1.3325x vs baseline; 1.3325x over previous
import jax
import jax.numpy as jnp
from jax import lax
from jax.experimental import pallas as pl
from jax.experimental.pallas import tpu as pltpu

N_DEV = 8
WNB = 1024
WDEPTH = 3


def kernel(x, w_mat):
    k_full, k_shard = x.shape
    _, n = w_mat.shape
    m_per = k_full // N_DEV
    npn = n // WNB
    n_pieces = N_DEV * npn

    def body(x_ref, w_ref, out_ref, xbf_ref, xg_ref, wbuf_ref, amax_ref,
             wdma_sems, send_sems, recv_sems, a_send_sems, a_recv_sems):
        my = lax.axis_index("i")

        sends = []
        for d in range(1, N_DEV):
            dst = lax.rem(my + d, N_DEV)
            rows = pl.ds(dst * m_per, m_per)
            xbf_ref[rows, :] = x_ref[rows, :].astype(jnp.bfloat16)
            rdma = pltpu.make_async_remote_copy(
                src_ref=xbf_ref.at[rows, :],
                dst_ref=xg_ref.at[:, pl.ds(my * k_shard, k_shard)],
                send_sem=send_sems.at[d],
                recv_sem=recv_sems.at[my],
                device_id=(dst,),
                device_id_type=pl.DeviceIdType.MESH,
            )
            rdma.start()
            sends.append(rdma)

        xg_ref[:, pl.ds(my * k_shard, k_shard)] = (
            x_ref[pl.ds(my * m_per, m_per), :].astype(jnp.bfloat16)
        )

        def w_dma(p, slot):
            d, nh = divmod(p, npn)
            src = lax.rem(my + d, N_DEV)
            return pltpu.make_async_copy(
                w_ref.at[pl.ds(src * k_shard, k_shard),
                         pl.ds(nh * WNB, WNB)],
                wbuf_ref.at[slot],
                wdma_sems.at[slot],
            )

        for p in range(min(2, n_pieces)):
            w_dma(p, p % WDEPTH).start()

        for p in range(n_pieces):
            d, nh = divmod(p, npn)
            src = lax.rem(my + d, N_DEV)
            if nh == 0 and d > 0:
                recv = pltpu.make_async_remote_copy(
                    src_ref=xbf_ref.at[pl.ds(0, m_per), :],
                    dst_ref=xg_ref.at[:, pl.ds(src * k_shard, k_shard)],
                    send_sem=send_sems.at[d],
                    recv_sem=recv_sems.at[src],
                    device_id=(my,),
                    device_id_type=pl.DeviceIdType.MESH,
                )
                recv.wait_recv()
            if p + 2 < n_pieces:
                w_dma(p + 2, (p + 2) % WDEPTH).start()
            w_dma(p, p % WDEPTH).wait()
            wb = wbuf_ref[p % WDEPTH].astype(jnp.bfloat16)
            partial = jnp.dot(
                xg_ref[:, pl.ds(src * k_shard, k_shard)],
                wb,
                preferred_element_type=jnp.float32,
            )
            cols = slice(nh * WNB, (nh + 1) * WNB)
            if d == 0:
                out_ref[:, cols] = partial
            else:
                out_ref[:, cols] += partial

        amax = jnp.float32(0.0)
        for nb in range(0, n, WNB):
            y = jnp.maximum(out_ref[:, nb:nb + WNB], 0.0)
            out_ref[:, nb:nb + WNB] = y
            amax = jnp.maximum(amax, jnp.max(y))

        amax_ref[pl.ds(my, 1), :] = jnp.full((1, 128), amax, jnp.float32)
        a_sends = []
        for d in range(1, N_DEV):
            dst = lax.rem(my + d, N_DEV)
            rdma = pltpu.make_async_remote_copy(
                src_ref=amax_ref.at[pl.ds(my, 1), :],
                dst_ref=amax_ref.at[pl.ds(my, 1), :],
                send_sem=a_send_sems.at[d],
                recv_sem=a_recv_sems.at[my],
                device_id=(dst,),
                device_id_type=pl.DeviceIdType.MESH,
            )
            rdma.start()
            a_sends.append(rdma)
        for d in range(1, N_DEV):
            src = lax.rem(my + d, N_DEV)
            recv = pltpu.make_async_remote_copy(
                src_ref=amax_ref.at[pl.ds(src, 1), :],
                dst_ref=amax_ref.at[pl.ds(src, 1), :],
                send_sem=a_send_sems.at[d],
                recv_sem=a_recv_sems.at[src],
                device_id=(my,),
                device_id_type=pl.DeviceIdType.MESH,
            )
            recv.wait_recv()

        gmax = jnp.max(amax_ref[:, 0])
        scale = gmax / 127.0
        for nb in range(0, n, WNB):
            q = jnp.clip(jnp.round(out_ref[:, nb:nb + WNB] / scale),
                         -127.0, 127.0)
            out_ref[:, nb:nb + WNB] = q * scale

        for r in sends:
            r.wait_send()
        for r in a_sends:
            r.wait_send()

    return pl.pallas_call(
        body,
        out_shape=jax.ShapeDtypeStruct((m_per, n), jnp.float32),
        in_specs=[
            pl.BlockSpec(memory_space=pltpu.MemorySpace.VMEM),
            pl.BlockSpec(memory_space=pl.ANY),
        ],
        out_specs=pl.BlockSpec(memory_space=pltpu.MemorySpace.VMEM),
        scratch_shapes=[
            pltpu.VMEM((k_full, k_shard), jnp.bfloat16),
            pltpu.VMEM((m_per, k_full), jnp.bfloat16),
            pltpu.VMEM((WDEPTH, k_shard, WNB), jnp.float32),
            pltpu.VMEM((N_DEV, 128), jnp.float32),
            pltpu.SemaphoreType.DMA((WDEPTH,)),
            pltpu.SemaphoreType.DMA((N_DEV,)),
            pltpu.SemaphoreType.DMA((N_DEV,)),
            pltpu.SemaphoreType.DMA((N_DEV,)),
            pltpu.SemaphoreType.DMA((N_DEV,)),
        ],
    )(x, w_mat)


# device time: 98030 ns/iter; 1.5661x vs baseline; 1.1754x over previous
import jax
import jax.numpy as jnp
from jax import lax
from jax.experimental import pallas as pl
from jax.experimental.pallas import tpu as pltpu

N_DEV = 8
NP = 1024
NC = 512
WDEPTH = 4
KG = 2


def make_kernel(variant="full"):

    def kernel(x, w_mat):
        return _kernel(x, w_mat, variant)

    return kernel


def _kernel(x, w_mat, variant):
    k_full, k_shard = x.shape
    _, n = w_mat.shape
    m_per = k_full // N_DEV
    n_panels = n // NP

    do_comm = variant not in ("nocomm", "compute")
    do_stream = variant not in ("nostream", "compute")
    do_epi = variant == "full"

    def body(x_ref, w_ref, out_ref, xbf_ref, xg_ref, wbuf_ref,
             wk_ref, amax_ref, wdma_sems, send_sems, recv_sems,
             a_send_sems, a_recv_sems):
        my = lax.axis_index("i")

        sends = []
        for e in range(1, N_DEV):
            dst = lax.rem(my + e, N_DEV)
            rows = pl.ds(dst * m_per, m_per)
            xbf_ref[rows, :] = x_ref[rows, :].astype(jnp.bfloat16)
            if not do_comm:
                xg_ref[:, e * k_shard:(e + 1) * k_shard] = xbf_ref[rows, :]
                continue
            rdma = pltpu.make_async_remote_copy(
                src_ref=xbf_ref.at[rows, :],
                dst_ref=xg_ref.at[:, e * k_shard:(e + 1) * k_shard],
                send_sem=send_sems.at[e],
                recv_sem=recv_sems.at[e],
                device_id=(dst,),
                device_id_type=pl.DeviceIdType.MESH,
            )
            rdma.start()
            sends.append(rdma)

        xg_ref[:, 0:k_shard] = (
            x_ref[pl.ds(my * m_per, m_per), :].astype(jnp.bfloat16)
        )

        def w_dma(q, slot):
            grp, rest = divmod(q, n_panels * KG)
            panel, a_local = divmod(rest, KG)
            a = grp * KG + a_local
            src = lax.rem(my - a + N_DEV, N_DEV)
            return pltpu.make_async_copy(
                w_ref.at[pl.ds(src * k_shard, k_shard),
                         pl.ds(panel * NP, NP)],
                wbuf_ref.at[slot],
                wdma_sems.at[slot],
            )

        n_groups = N_DEV // KG
        n_pieces = n_groups * n_panels * KG

        def recv_wait(a):
            recv = pltpu.make_async_remote_copy(
                src_ref=xbf_ref.at[pl.ds(0, m_per), :],
                dst_ref=xg_ref.at[:, a * k_shard:(a + 1) * k_shard],
                send_sem=send_sems.at[a],
                recv_sem=recv_sems.at[a],
                device_id=(my,),
                device_id_type=pl.DeviceIdType.MESH,
            )
            recv.wait_recv()

        amax = jnp.float32(0.0)
        if do_stream:
            for j in range(WDEPTH - 1):
                w_dma(j, j % WDEPTH).start()
        q = 0
        for grp in range(n_groups):
            for panel in range(n_panels):
                for a_local in range(KG):
                    if do_stream:
                        if q + WDEPTH - 1 < n_pieces:
                            w_dma(q + WDEPTH - 1,
                                  (q + WDEPTH - 1) % WDEPTH).start()
                        w_dma(q, q % WDEPTH).wait()
                    rows = slice(a_local * k_shard, (a_local + 1) * k_shard)
                    for c in range(NP // NC):
                        wk_ref[panel % 2, rows, c * NC:(c + 1) * NC] = (
                            wbuf_ref[q % WDEPTH, :, c * NC:(c + 1) * NC]
                            .astype(jnp.bfloat16)
                        )
                    q += 1
                if do_comm and panel == 0:
                    for a in range(max(1, grp * KG), (grp + 1) * KG):
                        recv_wait(a)
                xga = xg_ref[:, grp * KG * k_shard:(grp + 1) * KG * k_shard]
                for c in range(NP // NC):
                    partial = jnp.dot(
                        xga,
                        wk_ref[panel % 2, :, c * NC:(c + 1) * NC],
                        preferred_element_type=jnp.float32,
                    )
                    cols = slice(panel * NP + c * NC, panel * NP + (c + 1) * NC)
                    if grp == 0:
                        out_ref[:, cols] = partial
                    elif grp < n_groups - 1:
                        out_ref[:, cols] += partial
                    else:
                        v = out_ref[:, cols] + partial
                        y = jnp.maximum(v, 0.0)
                        out_ref[:, cols] = y
                        amax = jnp.maximum(amax, jnp.max(y))

        if do_epi:
            amax_ref[pl.ds(my, 1), :] = jnp.full((1, 128), amax, jnp.float32)
            a_sends = []
            for d in range(1, N_DEV):
                dst = lax.rem(my + d, N_DEV)
                rdma = pltpu.make_async_remote_copy(
                    src_ref=amax_ref.at[pl.ds(my, 1), :],
                    dst_ref=amax_ref.at[pl.ds(my, 1), :],
                    send_sem=a_send_sems.at[d],
                    recv_sem=a_recv_sems.at[my],
                    device_id=(dst,),
                    device_id_type=pl.DeviceIdType.MESH,
                )
                rdma.start()
                a_sends.append(rdma)
            for d in range(1, N_DEV):
                src = lax.rem(my + d, N_DEV)
                recv = pltpu.make_async_remote_copy(
                    src_ref=amax_ref.at[pl.ds(src, 1), :],
                    dst_ref=amax_ref.at[pl.ds(src, 1), :],
                    send_sem=a_send_sems.at[d],
                    recv_sem=a_recv_sems.at[src],
                    device_id=(my,),
                    device_id_type=pl.DeviceIdType.MESH,
                )
                recv.wait_recv()

            gmax = jnp.max(amax_ref[:, 0])
            scale = gmax / 127.0
            for nb in range(0, n, NC):
                qv = jnp.clip(jnp.round(out_ref[:, nb:nb + NC] / scale),
                              -127.0, 127.0)
                out_ref[:, nb:nb + NC] = qv * scale

            for r in a_sends:
                r.wait_send()

        for r in sends:
            r.wait_send()

    return pl.pallas_call(
        body,
        out_shape=jax.ShapeDtypeStruct((m_per, n), jnp.float32),
        in_specs=[
            pl.BlockSpec(memory_space=pltpu.MemorySpace.VMEM),
            pl.BlockSpec(memory_space=pl.ANY),
        ],
        out_specs=pl.BlockSpec(memory_space=pltpu.MemorySpace.VMEM),
        scratch_shapes=[
            pltpu.VMEM((k_full, k_shard), jnp.bfloat16),
            pltpu.VMEM((m_per, k_full), jnp.bfloat16),
            pltpu.VMEM((WDEPTH, k_shard, NP), jnp.float32),
            pltpu.VMEM((2, KG * k_shard, NP), jnp.bfloat16),
            pltpu.VMEM((N_DEV, 128), jnp.float32),
            pltpu.SemaphoreType.DMA((WDEPTH,)),
            pltpu.SemaphoreType.DMA((N_DEV,)),
            pltpu.SemaphoreType.DMA((N_DEV,)),
            pltpu.SemaphoreType.DMA((N_DEV,)),
            pltpu.SemaphoreType.DMA((N_DEV,)),
        ],
        compiler_params=pltpu.CompilerParams(
            vmem_limit_bytes=60 * 1024 * 1024,
        ),
    )(x, w_mat)


kernel = make_kernel("full")


# device time: 96235 ns/iter; 1.5953x vs baseline; 1.0187x over previous
import jax
import jax.numpy as jnp
from jax import lax
from jax.experimental import pallas as pl
from jax.experimental.pallas import tpu as pltpu

N_DEV = 8
NP = 1024
NC = 512
WDEPTH = 5
KG = 2


def make_kernel(variant="full"):

    def kernel(x, w_mat):
        return _kernel(x, w_mat, variant)

    return kernel


def _kernel(x, w_mat, variant):
    k_full, k_shard = x.shape
    _, n = w_mat.shape
    m_per = k_full // N_DEV
    n_panels = n // NP

    do_comm = variant not in ("nocomm", "compute")
    do_stream = variant not in ("nostream", "compute")
    do_epi = variant == "full"

    def body(x_ref, w_ref, out_ref, xbf_ref, xg_ref, wbuf_ref,
             wk_ref, amax_ref, wdma_sems, send_sems, recv_sems,
             a_send_sems, a_recv_sems):
        my = lax.axis_index("i")

        sends = []
        for e in range(1, N_DEV):
            dst = lax.rem(my + e, N_DEV)
            rows = pl.ds(dst * m_per, m_per)
            xbf_ref[rows, :] = x_ref[rows, :].astype(jnp.bfloat16)
            if not do_comm:
                xg_ref[:, e * k_shard:(e + 1) * k_shard] = xbf_ref[rows, :]
                continue
            rdma = pltpu.make_async_remote_copy(
                src_ref=xbf_ref.at[rows, :],
                dst_ref=xg_ref.at[:, e * k_shard:(e + 1) * k_shard],
                send_sem=send_sems.at[e],
                recv_sem=recv_sems.at[e],
                device_id=(dst,),
                device_id_type=pl.DeviceIdType.MESH,
            )
            rdma.start()
            sends.append(rdma)

        xg_ref[:, 0:k_shard] = (
            x_ref[pl.ds(my * m_per, m_per), :].astype(jnp.bfloat16)
        )

        def w_dma(q, slot):
            grp, rest = divmod(q, n_panels * KG)
            panel, a_local = divmod(rest, KG)
            a = grp * KG + a_local
            src = lax.rem(my - a + N_DEV, N_DEV)
            return pltpu.make_async_copy(
                w_ref.at[pl.ds(src * k_shard, k_shard),
                         pl.ds(panel * NP, NP)],
                wbuf_ref.at[slot],
                wdma_sems.at[slot],
            )

        n_groups = N_DEV // KG
        n_pieces = n_groups * n_panels * KG

        def recv_wait(a):
            recv = pltpu.make_async_remote_copy(
                src_ref=xbf_ref.at[pl.ds(0, m_per), :],
                dst_ref=xg_ref.at[:, a * k_shard:(a + 1) * k_shard],
                send_sem=send_sems.at[a],
                recv_sem=recv_sems.at[a],
                device_id=(my,),
                device_id_type=pl.DeviceIdType.MESH,
            )
            recv.wait_recv()

        amax = jnp.float32(0.0)
        if do_stream:
            for j in range(WDEPTH - 1):
                w_dma(j, j % WDEPTH).start()
        q = 0
        for grp in range(n_groups):
            for panel in range(n_panels):
                for a_local in range(KG):
                    if do_stream:
                        if q + WDEPTH - 1 < n_pieces:
                            w_dma(q + WDEPTH - 1,
                                  (q + WDEPTH - 1) % WDEPTH).start()
                        w_dma(q, q % WDEPTH).wait()
                    rows = slice(a_local * k_shard, (a_local + 1) * k_shard)
                    for c in range(NP // NC):
                        wk_ref[panel % 2, rows, c * NC:(c + 1) * NC] = (
                            wbuf_ref[q % WDEPTH, :, c * NC:(c + 1) * NC]
                            .astype(jnp.bfloat16)
                        )
                    q += 1
                if do_comm and panel == 0:
                    for a in range(max(1, grp * KG), (grp + 1) * KG):
                        recv_wait(a)
                xga = xg_ref[:, grp * KG * k_shard:(grp + 1) * KG * k_shard]
                for c in range(NP // NC):
                    partial = jnp.dot(
                        xga,
                        wk_ref[panel % 2, :, c * NC:(c + 1) * NC],
                        preferred_element_type=jnp.float32,
                    )
                    cols = slice(panel * NP + c * NC, panel * NP + (c + 1) * NC)
                    if grp == 0:
                        out_ref[:, cols] = partial
                    elif grp < n_groups - 1:
                        out_ref[:, cols] += partial
                    else:
                        v = out_ref[:, cols] + partial
                        y = jnp.maximum(v, 0.0)
                        out_ref[:, cols] = y
                        amax = jnp.maximum(amax, jnp.max(y))

        if do_epi:
            amax_ref[pl.ds(my, 1), :] = jnp.full((1, 128), amax, jnp.float32)
            a_sends = []
            for d in range(1, N_DEV):
                dst = lax.rem(my + d, N_DEV)
                rdma = pltpu.make_async_remote_copy(
                    src_ref=amax_ref.at[pl.ds(my, 1), :],
                    dst_ref=amax_ref.at[pl.ds(my, 1), :],
                    send_sem=a_send_sems.at[d],
                    recv_sem=a_recv_sems.at[my],
                    device_id=(dst,),
                    device_id_type=pl.DeviceIdType.MESH,
                )
                rdma.start()
                a_sends.append(rdma)
            for d in range(1, N_DEV):
                src = lax.rem(my + d, N_DEV)
                recv = pltpu.make_async_remote_copy(
                    src_ref=amax_ref.at[pl.ds(src, 1), :],
                    dst_ref=amax_ref.at[pl.ds(src, 1), :],
                    send_sem=a_send_sems.at[d],
                    recv_sem=a_recv_sems.at[src],
                    device_id=(my,),
                    device_id_type=pl.DeviceIdType.MESH,
                )
                recv.wait_recv()

            gmax = jnp.max(amax_ref[:, 0])
            scale = gmax / 127.0
            for nb in range(0, n, NC):
                qv = jnp.clip(jnp.round(out_ref[:, nb:nb + NC] / scale),
                              -127.0, 127.0)
                out_ref[:, nb:nb + NC] = qv * scale

            for r in a_sends:
                r.wait_send()

        for r in sends:
            r.wait_send()

    return pl.pallas_call(
        body,
        out_shape=jax.ShapeDtypeStruct((m_per, n), jnp.float32),
        in_specs=[
            pl.BlockSpec(memory_space=pltpu.MemorySpace.VMEM),
            pl.BlockSpec(memory_space=pl.ANY),
        ],
        out_specs=pl.BlockSpec(memory_space=pltpu.MemorySpace.VMEM),
        scratch_shapes=[
            pltpu.VMEM((k_full, k_shard), jnp.bfloat16),
            pltpu.VMEM((m_per, k_full), jnp.bfloat16),
            pltpu.VMEM((WDEPTH, k_shard, NP), jnp.float32),
            pltpu.VMEM((2, KG * k_shard, NP), jnp.bfloat16),
            pltpu.VMEM((N_DEV, 128), jnp.float32),
            pltpu.SemaphoreType.DMA((WDEPTH,)),
            pltpu.SemaphoreType.DMA((N_DEV,)),
            pltpu.SemaphoreType.DMA((N_DEV,)),
            pltpu.SemaphoreType.DMA((N_DEV,)),
            pltpu.SemaphoreType.DMA((N_DEV,)),
        ],
        compiler_params=pltpu.CompilerParams(
            vmem_limit_bytes=60 * 1024 * 1024,
        ),
    )(x, w_mat)


kernel = make_kernel("full")
